# baseline (speedup 1.0000x reference)
import jax
import jax.numpy as jnp
from jax.experimental import pallas as pl

S, B, W = 50, 1024, 21
V, EC, EW, K = 96, 50, 256, 5
N = S * B
P = W + 2
T = P - K + 1
NB = 1280


def _body(idx_ref, tbl_ref, wk_ref, cb_ref, wp_ref, bp_ref, wg_ref, bg_ref,
          out_ref):
    idx = idx_ref[...][..., None]
    oh = (idx == jax.lax.broadcasted_iota(jnp.int16, (P, NB, V), 2)
          ).astype(jnp.bfloat16).reshape(P * NB, V)
    emb = jax.lax.dot_general(
        oh, tbl_ref[...], (((1,), (0,)), ((), ())),
        preferred_element_type=jnp.float32).astype(jnp.bfloat16
                                                   ).reshape(P, NB, EC)
    xwin = jnp.concatenate([emb[k:k + T] for k in range(K)],
                           axis=2).reshape(T * NB, K * EC)
    acc = jax.lax.dot_general(
        xwin, wk_ref[...], (((1,), (0,)), ((), ())),
        preferred_element_type=jnp.float32)
    h = jnp.maximum(jnp.max(acc.reshape(T, NB, EW), axis=0) + cb_ref[...], 0.0)
    xp = jnp.maximum(
        jax.lax.dot_general(h, wp_ref[...], (((1,), (0,)), ((), ())),
                            preferred_element_type=jnp.float32) + bp_ref[...],
        0.0)
    xg = jax.nn.sigmoid(
        jax.lax.dot_general(h, wg_ref[...], (((1,), (0,)), ((), ())),
                            preferred_element_type=jnp.float32) + bg_ref[...])
    out_ref[...] = xg * xp + (1.0 - xg) * h


@jax.jit
def _run(idxp, tbl0, wk, cb, wpT, bp, wgT, bg):
    full = lambda shape: pl.BlockSpec(shape, lambda i: (0,) * len(shape))
    return pl.pallas_call(
        _body,
        grid=(N // NB,),
        in_specs=[
            pl.BlockSpec((P, NB), lambda i: (0, i)),
            full((V, EC)),
            full((K * EC, EW)),
            full((1, EW)),
            full((EW, EW)),
            full((1, EW)),
            full((EW, EW)),
            full((1, EW)),
        ],
        out_specs=pl.BlockSpec((NB, EW), lambda i: (i, 0)),
        out_shape=jax.ShapeDtypeStruct((N, EW), jnp.float32),
    )(idxp, tbl0, wk, cb, wpT, bp, wgT, bg)


def kernel(input, table, conv_w, conv_b, w_proj, b_proj, w_gate, b_gate):
    idxp = jnp.pad(input.reshape(N, W), ((0, 0), (1, 1))).T.astype(jnp.int16)
    tbl0 = table.at[0].set(0.0).astype(jnp.bfloat16)
    wk = conv_w.transpose(2, 1, 0).reshape(K * EC, EW).astype(jnp.bfloat16)
    out = _run(idxp, tbl0, wk, conv_b.reshape(1, EW), w_proj.T,
               b_proj.reshape(1, EW), w_gate.T, b_gate.reshape(1, EW))
    return out.reshape(S, B, EW)
